# baseline (speedup 1.0000x reference)
import functools

import jax
import jax.numpy as jnp
from jax import lax
from jax.experimental import pallas as pl
from jax.experimental.pallas import tpu as pltpu
from jax.experimental.pallas import tpu_sc as plsc

D = 32
B1 = 4096
B2 = 200
PACK = 4
NW = 32
CHUNK = 400


def _build_gather(tr0, ntr):
  ng = 2 * ntr
  mesh = plsc.VectorSubcoreMesh(core_axis_name="c", subcore_axis_name="s")

  @functools.partial(
      pl.kernel,
      mesh=mesh,
      compiler_params=pltpu.CompilerParams(use_tc_tiling_on_sc=False),
      out_type=jax.ShapeDtypeStruct((ng * B1, PACK * D), jnp.float32),
      scratch_types=[
          pltpu.VMEM((ntr, 8 * 128), jnp.int32),
          pltpu.VMEM((4, PACK * 128, D), jnp.float32),
          pltpu.SemaphoreType.DMA,
          pltpu.SemaphoreType.DMA,
          pltpu.SemaphoreType.DMA,
          pltpu.SemaphoreType.DMA,
          pltpu.SemaphoreType.DMA,
      ],
  )
  def emb(idx_hbm, table_hbm, out_hbm, idx_v, g_v, sem_g, sw0, sw1, sw2, sw3):
    w = lax.axis_index("s") * 2 + lax.axis_index("c")
    pltpu.sync_copy(idx_hbm.at[pl.ds(tr0, ntr), w], idx_v)
    sem_w = (sw0, sw1, sw2, sw3)

    def gather(g):
      return pltpu.async_copy(
          table_hbm.at[idx_v.at[g // 2, pl.ds((g % 2) * 512, PACK * 128)]],
          g_v.at[g % 4], sem_g)

    def writes(g):
      return [
          pltpu.async_copy(
              g_v.at[g % 4, pl.ds(m * 128, 128)],
              out_hbm.at[pl.ds(g * B1 + 128 * w, 128), pl.ds(m * D, D)],
              sem_w[g % 4])
          for m in range(PACK)
      ]

    g_descs = [None] * ng
    w_descs = [None] * ng
    g_descs[0] = gather(0)
    g_descs[1] = gather(1)
    for g in range(ng):
      if g >= 2:
        for d in w_descs[g - 2]:
          d.wait()
      if g + 2 < ng:
        g_descs[g + 2] = gather(g + 2)
      g_descs[g].wait()
      w_descs[g] = writes(g)
    for g in (ng - 2, ng - 1):
      for d in w_descs[g]:
        d.wait()

  return emb


def _tc_transpose(y, g0, ng, prev=None):
  NC = B1 // 128

  def body(*refs):
    y_ref, o_ref = refs[0], refs[-1]
    ident = jnp.eye(128, dtype=jnp.float32)
    blk = y_ref[...].reshape(NC, 128, 128)
    t = lax.dot_general(
        blk, ident, (((1,), (0,)), ((), ())),
        preferred_element_type=jnp.float32)
    t5 = t.reshape(NC, PACK, D // 8, 8, 128)
    o_ref[...] = t5.transpose(1, 2, 0, 3, 4)

  in_specs = [pl.BlockSpec((B1, 128), lambda g: (g, 0))]
  operands = [y]
  kwargs = {}
  if prev is not None:
    in_specs.append(pl.BlockSpec(memory_space=pl.ANY))
    operands.append(prev)
    kwargs["input_output_aliases"] = {1: 0}
  return pl.pallas_call(
      body,
      grid=(ng,),
      in_specs=in_specs,
      out_specs=pl.BlockSpec((PACK, D // 8, NC, 8, 128),
                             lambda g: (g + g0, 0, 0, 0, 0)),
      out_shape=jax.ShapeDtypeStruct((B2, D // 8, NC, 8, 128),
                                     jnp.float32),
      **kwargs,
  )(*operands)


@jax.jit
def kernel(x, table):
  xi = x.astype(jnp.int32)
  x5c = (xi.T.reshape(B2 // 8, 8, B1 // 128, 128).swapaxes(1, 2)
         .reshape(B2 // 8, B1 // 128, 8 * 128))
  y1 = _build_gather(0, 13)(x5c, table)
  y2 = _build_gather(13, 12)(x5c, table)
  h1 = _tc_transpose(y1, 0, 26)
  out5 = _tc_transpose(y2, 26, 24, prev=h1)
  return out5.transpose(2, 4, 0, 1, 3).reshape(B1, B2, D)


# speedup vs baseline: 1.0025x; 1.0025x over previous
import functools

import jax
import jax.numpy as jnp
from jax import lax
from jax.experimental import pallas as pl
from jax.experimental.pallas import tpu as pltpu
from jax.experimental.pallas import tpu_sc as plsc

D = 32
B1 = 4096
B2 = 200
PACK = 4
NW = 32
CHUNK = 400


def _build_gather():
  NG = B2 // PACK
  mesh = plsc.VectorSubcoreMesh(core_axis_name="c", subcore_axis_name="s")

  @functools.partial(
      pl.kernel,
      mesh=mesh,
      compiler_params=pltpu.CompilerParams(use_tc_tiling_on_sc=False),
      out_type=jax.ShapeDtypeStruct(((B1 * B2) // PACK, PACK * D),
                                    jnp.float32),
      scratch_types=[
          pltpu.VMEM((B2 // 8, 8 * 128), jnp.int32),
          pltpu.VMEM((4, PACK * 128, D), jnp.float32),
          pltpu.SemaphoreType.DMA,
          pltpu.SemaphoreType.DMA,
          pltpu.SemaphoreType.DMA,
          pltpu.SemaphoreType.DMA,
          pltpu.SemaphoreType.DMA,
      ],
  )
  def emb(idx_hbm, table_hbm, out_hbm, idx_v, g_v, sem_g, sw0, sw1, sw2, sw3):
    w = lax.axis_index("s") * 2 + lax.axis_index("c")
    pltpu.sync_copy(idx_hbm.at[:, w], idx_v)
    sem_w = (sw0, sw1, sw2, sw3)

    def gather(g):
      return pltpu.async_copy(
          table_hbm.at[idx_v.at[g // 2, pl.ds((g % 2) * 512, PACK * 128)]],
          g_v.at[g % 4], sem_g)

    def writes(g):
      return [
          pltpu.async_copy(
              g_v.at[g % 4, pl.ds(m * 128, 128)],
              out_hbm.at[pl.ds(g * B1 + 128 * w, 128), pl.ds(m * D, D)],
              sem_w[g % 4])
          for m in range(PACK)
      ]

    g_descs = [None] * NG
    w_descs = [None] * NG
    g_descs[0] = gather(0)
    g_descs[1] = gather(1)
    for g in range(NG):
      if g >= 2:
        for d in w_descs[g - 2]:
          d.wait()
      if g + 2 < NG:
        g_descs[g + 2] = gather(g + 2)
      g_descs[g].wait()
      w_descs[g] = writes(g)
    for g in (NG - 2, NG - 1):
      for d in w_descs[g]:
        d.wait()

  return emb


def _tc_transpose(y):
  NC = B1 // 128

  def body(y_ref, o_ref):
    ident = jnp.eye(128, dtype=jnp.float32)
    blk = y_ref[...].reshape(NC, 128, 128)
    t = lax.dot_general(
        blk, ident, (((1,), (0,)), ((), ())),
        preferred_element_type=jnp.float32)
    t5 = t.reshape(NC, PACK, D // 8, 8, 128)
    o_ref[...] = t5.transpose(1, 2, 0, 3, 4)

  return pl.pallas_call(
      body,
      grid=(B2 // PACK,),
      in_specs=[pl.BlockSpec((B1, 128), lambda g: (g, 0))],
      out_specs=pl.BlockSpec((PACK, D // 8, NC, 8, 128),
                             lambda g: (g, 0, 0, 0, 0)),
      out_shape=jax.ShapeDtypeStruct((B2, D // 8, NC, 8, 128),
                                     jnp.float32),
  )(y)


@jax.jit
def kernel(x, table):
  xi = x.astype(jnp.int32)
  x5c = (xi.T.reshape(B2 // 8, 8, B1 // 128, 128).swapaxes(1, 2)
         .reshape(B2 // 8, B1 // 128, 8 * 128))
  y = _build_gather()(x5c, table)
  out5 = _tc_transpose(y)
  return out5.transpose(2, 4, 0, 1, 3).reshape(B1, B2, D)
